# baseline (speedup 1.0000x reference)
import jax
import jax.numpy as jnp
from jax.experimental import pallas as pl

_HEIGHT = 4096
_WIDTH = 4096
_BATCH = 2048
_NBLK = 128


def _decode_matmul_kernel(x_ref, code_ref, mm_ref, out_ref):
    code = code_ref[...]
    mins = mm_ref[:, 0:1]
    maxs = mm_ref[:, 1:2]
    w = (mins * (code == 1).astype(jnp.bfloat16)
         + maxs * (code == 2).astype(jnp.bfloat16))
    out_ref[...] = jax.lax.dot_general(
        x_ref[...],
        w,
        (((1,), (1,)), ((), ())),
        preferred_element_type=jnp.float32,
    ).astype(jnp.bfloat16)


def kernel(x, w_tern, ter_minmax):
    mm = ter_minmax.reshape(_HEIGHT, 2)
    nj = _HEIGHT // _NBLK
    return pl.pallas_call(
        _decode_matmul_kernel,
        grid=(nj,),
        in_specs=[
            pl.BlockSpec((_BATCH, _WIDTH), lambda j: (0, 0)),
            pl.BlockSpec((_NBLK, _WIDTH), lambda j: (j, 0)),
            pl.BlockSpec((_NBLK, 2), lambda j: (j, 0)),
        ],
        out_specs=pl.BlockSpec((_BATCH, _NBLK), lambda j: (0, j)),
        out_shape=jax.ShapeDtypeStruct((_BATCH, _HEIGHT), jnp.bfloat16),
    )(x, w_tern, mm)


# speedup vs baseline: 1.6972x; 1.6972x over previous
import jax
import jax.numpy as jnp
from jax.experimental import pallas as pl

_HEIGHT = 4096
_WIDTH = 4096
_BATCH = 2048
_NBLK = 512
_MBLK = 256


def _decode_matmul_kernel(x_ref, code_ref, mm_ref, out_ref):
    code = code_ref[...]
    mins = mm_ref[:, 0:1]
    maxs = mm_ref[:, 1:2]
    w = (mins * (code == 1).astype(jnp.bfloat16)
         + maxs * (code == 2).astype(jnp.bfloat16))
    for m in range(0, _BATCH, _MBLK):
        out_ref[m:m + _MBLK, :] = jax.lax.dot_general(
            x_ref[m:m + _MBLK, :],
            w,
            (((1,), (1,)), ((), ())),
            preferred_element_type=jnp.float32,
        ).astype(jnp.bfloat16)


def kernel(x, w_tern, ter_minmax):
    mm = ter_minmax.reshape(_HEIGHT, 2)
    nj = _HEIGHT // _NBLK
    return pl.pallas_call(
        _decode_matmul_kernel,
        grid=(nj,),
        in_specs=[
            pl.BlockSpec((_BATCH, _WIDTH), lambda j: (0, 0)),
            pl.BlockSpec((_NBLK, _WIDTH), lambda j: (j, 0)),
            pl.BlockSpec((_NBLK, 2), lambda j: (j, 0)),
        ],
        out_specs=pl.BlockSpec((_BATCH, _NBLK), lambda j: (0, j)),
        out_shape=jax.ShapeDtypeStruct((_BATCH, _HEIGHT), jnp.bfloat16),
    )(x, w_tern, mm)


# speedup vs baseline: 1.8091x; 1.0659x over previous
import jax
import jax.numpy as jnp
from jax.experimental import pallas as pl

_HEIGHT = 4096
_WIDTH = 4096
_BATCH = 2048
_NBLK = 512
_MBLK = 512


def _decode_matmul_kernel(x_ref, code_ref, mm_ref, out_ref):
    code = code_ref[...]
    mins = mm_ref[:, 0:1]
    maxs = mm_ref[:, 1:2]
    w = (mins * (code == 1).astype(jnp.bfloat16)
         + maxs * (code == 2).astype(jnp.bfloat16))
    for m in range(0, _BATCH, _MBLK):
        out_ref[m:m + _MBLK, :] = jax.lax.dot_general(
            x_ref[m:m + _MBLK, :],
            w,
            (((1,), (1,)), ((), ())),
            preferred_element_type=jnp.float32,
        ).astype(jnp.bfloat16)


def kernel(x, w_tern, ter_minmax):
    mm = ter_minmax.reshape(_HEIGHT, 2)
    nj = _HEIGHT // _NBLK
    return pl.pallas_call(
        _decode_matmul_kernel,
        grid=(nj,),
        in_specs=[
            pl.BlockSpec((_BATCH, _WIDTH), lambda j: (0, 0)),
            pl.BlockSpec((_NBLK, _WIDTH), lambda j: (j, 0)),
            pl.BlockSpec((_NBLK, 2), lambda j: (j, 0)),
        ],
        out_specs=pl.BlockSpec((_BATCH, _NBLK), lambda j: (0, j)),
        out_shape=jax.ShapeDtypeStruct((_BATCH, _HEIGHT), jnp.bfloat16),
    )(x, w_tern, mm)


# speedup vs baseline: 1.8155x; 1.0035x over previous
import jax
import jax.numpy as jnp
from jax.experimental import pallas as pl

_HEIGHT = 4096
_WIDTH = 4096
_BATCH = 2048
_NBLK = 512
_MBLK = 1024


def _decode_matmul_kernel(x_ref, code_ref, mm_ref, out_ref):
    code = code_ref[...]
    mins = mm_ref[:, 0:1]
    maxs = mm_ref[:, 1:2]
    w = (mins * (code == 1).astype(jnp.bfloat16)
         + maxs * (code == 2).astype(jnp.bfloat16))
    for m in range(0, _BATCH, _MBLK):
        out_ref[m:m + _MBLK, :] = jax.lax.dot_general(
            x_ref[m:m + _MBLK, :],
            w,
            (((1,), (1,)), ((), ())),
            preferred_element_type=jnp.float32,
        ).astype(jnp.bfloat16)


def kernel(x, w_tern, ter_minmax):
    mm = ter_minmax.reshape(_HEIGHT, 2)
    nj = _HEIGHT // _NBLK
    return pl.pallas_call(
        _decode_matmul_kernel,
        grid=(nj,),
        in_specs=[
            pl.BlockSpec((_BATCH, _WIDTH), lambda j: (0, 0)),
            pl.BlockSpec((_NBLK, _WIDTH), lambda j: (j, 0)),
            pl.BlockSpec((_NBLK, 2), lambda j: (j, 0)),
        ],
        out_specs=pl.BlockSpec((_BATCH, _NBLK), lambda j: (0, j)),
        out_shape=jax.ShapeDtypeStruct((_BATCH, _HEIGHT), jnp.bfloat16),
    )(x, w_tern, mm)
